# baseline (speedup 1.0000x reference)
import functools

import jax
import jax.numpy as jnp
from jax import lax
from jax.experimental import pallas as pl
from jax.experimental.pallas import tpu as pltpu
from jax.experimental.pallas import tpu_sc as plsc

CELL = 8
THRESH = 0.015
NEG = float("-inf")
BIG = 3e7
SENT = -3e38

STRIP = 64
HALO = 3
TILE = STRIP + 2 * HALO
NSTRIPS = 256 // STRIP


def _pool_y(x, op, padval):
    T = x.shape[2]
    P = [x[0]]
    for p in range(1, 8):
        P.append(op(P[-1], x[p]))
    S = [None] * 8
    S[7] = x[7]
    for p in range(6, -1, -1):
        S[p] = op(S[p + 1], x[p])
    pad = jnp.full((8, 1, 256), padval, x.dtype)

    def up(a):
        return jnp.concatenate([pad, a[:, : T - 1, :]], axis=1)

    def down(a):
        return jnp.concatenate([a[:, 1:, :], pad], axis=1)

    outs = [op(P[cy + 4], up(S[cy + 4])) for cy in range(4)]
    outs += [op(S[cy - 4], down(P[cy - 4])) for cy in range(4, 8)]
    return jnp.stack(outs, axis=0)


def _pool_x(x, op, padval):
    T = x.shape[2]
    P = [x[:, 0]]
    for p in range(1, 8):
        P.append(op(P[-1], x[:, p]))
    S = [None] * 8
    S[7] = x[:, 7]
    for p in range(6, -1, -1):
        S[p] = op(S[p + 1], x[:, p])
    pad = jnp.full((8, T, 1), padval, x.dtype)

    def left(a):
        return jnp.concatenate([pad, a[:, :, :-1]], axis=2)

    def right(a):
        return jnp.concatenate([a[:, :, 1:], pad], axis=2)

    outs = [op(P[cx + 4], left(S[cx + 4])) for cx in range(4)]
    outs += [op(S[cx - 4], right(P[cx - 4])) for cx in range(4, 8)]
    return jnp.stack(outs, axis=1)


def _pool9(x):
    return _pool_x(_pool_y(x, jnp.maximum, NEG), jnp.maximum, NEG)


def _dilate9(m):
    mb = m.astype(jnp.bfloat16)
    return _pool_x(_pool_y(mb, jnp.maximum, NEG), jnp.maximum, NEG) > 0


def _body(b_ref, c_ref, out_ref, tail_ref, sm_s, sm_i):
    i = pl.program_id(0)

    @pl.when(i == 0)
    def _():
        tail_ref[...] = jnp.zeros_like(tail_ref)

    xa = tail_ref[...]
    xb = b_ref[...]
    xc = c_ref[:, :HALO, :]
    x = jnp.concatenate([xa, xb, xc], axis=1)
    tail_ref[...] = b_ref[:, STRIP - HALO :, :]
    mx = jnp.max(x, axis=0, keepdims=True)
    ex = jnp.exp(x - mx)
    denom = jnp.sum(ex, axis=0, keepdims=True)
    probs = ex[:64] / denom
    s = probs.reshape(8, 8, TILE, 256)

    g0 = i * STRIP - HALO
    rowid = jax.lax.broadcasted_iota(jnp.int32, (1, 1, TILE, 256), 2) + g0
    s = jnp.where((rowid >= 0) & (rowid < 256), s, NEG)

    mask = s == _pool9(s)
    for _ in range(2):
        supp = _dilate9(mask)
        supp_scores = jnp.where(supp, 0.0, s)
        new_max = supp_scores == _pool9(supp_scores)
        mask = mask | (new_max & jnp.logical_not(supp))
    nms = jnp.where(mask, s, 0.0)

    core = nms[:, :, HALO : HALO + STRIP, :]
    vals = jnp.where(core > THRESH, core, SENT)
    groups = []
    for hy in range(2):
        for hx in range(2):
            bv = bi = None
            for cy in range(hy * 4, hy * 4 + 4):
                for cx in range(hx * 4, hx * 4 + 4):
                    v = vals[cy, cx]
                    iconst = jnp.float32(cy * 2048 + cx)
                    if bv is None:
                        bv, bi = v, jnp.full_like(v, iconst)
                    else:
                        gt = v > bv
                        bv = jnp.where(gt, v, bv)
                        bi = jnp.where(gt, iconst, bi)
            groups.append((bv, bi))
    V4 = jnp.stack([g[0] for g in groups], axis=0)
    I4 = jnp.stack([g[1] for g in groups], axis=0)
    ir = jax.lax.broadcasted_iota(jnp.int32, (1, STRIP, 1), 1)
    ik = jax.lax.broadcasted_iota(jnp.int32, (1, 1, 256), 2)
    rowbase = (8 * (STRIP * i + ir) * 2048).astype(jnp.float32)
    colbase = (8 * ik).astype(jnp.float32)
    flatv = I4 + rowbase + colbase
    cand = []
    v = V4
    for _ in range(3):
        m = jnp.max(v)
        eqm = v == m
        sel = jnp.min(jnp.where(eqm, flatv, BIG))
        v = jnp.where(eqm & (flatv == sel), NEG, v)
        cand.append((m, sel))

    @pl.when(i == 0)
    def _():
        for t in range(3):
            sm_s[t] = jnp.float32(NEG)
            sm_i[t] = jnp.float32(BIG)

    pairs = [(sm_s[0], sm_i[0]), (sm_s[1], sm_i[1]), (sm_s[2], sm_i[2])] + cand
    top = []
    cur = pairs
    for _ in range(3):
        bs, bi = cur[0]
        for ss, si in cur[1:]:
            better = (ss > bs) | ((ss == bs) & (si < bi))
            bs = jnp.where(better, ss, bs)
            bi = jnp.where(better, si, bi)
        top.append((bs, bi))
        cur = [
            (
                jnp.where((ss == bs) & (si == bi), jnp.float32(NEG), ss),
                jnp.where((ss == bs) & (si == bi), jnp.float32(BIG), si),
            )
            for ss, si in cur
        ]
    for t in range(3):
        sm_s[t] = top[t][0]
        sm_i[t] = top[t][1]

    @pl.when(i == NSTRIPS - 1)
    def _():
        rI = jax.lax.broadcasted_iota(jnp.int32, (8, 128), 0)
        cI = jax.lax.broadcasted_iota(jnp.int32, (8, 128), 1)
        acc = jnp.zeros((8, 128), jnp.float32)
        entries = [(0, 0, top[0][0]), (0, 1, top[1][0]), (0, 2, top[2][0]),
                   (0, 3, top[0][1]), (0, 4, top[1][1]), (0, 5, top[2][1])]
        for rr, cc, val in entries:
            acc = acc + jnp.where((rI == rr) & (cI == cc), val, 0.0)
        out_ref[...] = acc


def _detector(det_p):
    return pl.pallas_call(
        _body,
        grid=(NSTRIPS,),
        in_specs=[
            pl.BlockSpec((65, STRIP, 256), lambda i: (0, i, 0)),
            pl.BlockSpec((65, STRIP, 256), lambda i: (0, i + 1, 0)),
        ],
        out_specs=pl.BlockSpec((8, 128), lambda i: (0, 0)),
        out_shape=jax.ShapeDtypeStruct((8, 128), jnp.float32),
        scratch_shapes=[
            pltpu.VMEM((65, HALO, 256), jnp.float32),
            pltpu.SMEM((8,), jnp.float32),
            pltpu.SMEM((8,), jnp.float32),
        ],
    )(det_p, det_p)


def _sc_finish(top16, cls_flat):
    mesh = plsc.VectorSubcoreMesh(core_axis_name="c", subcore_axis_name="s")

    @functools.partial(
        pl.kernel,
        mesh=mesh,
        out_type=jax.ShapeDtypeStruct((16,), jnp.float32),
        scratch_types=[
            pltpu.VMEM((16,), jnp.float32),
            pltpu.VMEM((16,), jnp.int32),
            pltpu.VMEM((16,), jnp.float32),
            pltpu.VMEM((16,), jnp.float32),
            pltpu.SemaphoreType.DMA,
        ],
    )
    def k(top_hbm, cls_hbm, out_hbm, top_v, idx_v, gat_v, out_v, sem):
        wid = lax.axis_index("s") * 2 + lax.axis_index("c")

        @pl.when(wid == 0)
        def _():
            pltpu.sync_copy(top_hbm, top_v)
            tv = top_v[...]
            scores = [tv[j] for j in range(3)]
            scores = [jnp.where(sp == SENT, jnp.float32(NEG), sp)
                      for sp in scores]
            idx = [tv[3 + j].astype(jnp.int32) for j in range(3)]
            r = [idx[j] // 2048 for j in range(3)]
            c = [idx[j] % 2048 for j in range(3)]
            lane = lax.iota(jnp.int32, 16)
            gidx = jnp.zeros((16,), jnp.int32)
            for j in range(3):
                base = (r[j] // CELL) * 256 + (c[j] // CELL)
                for chn in range(4):
                    gidx = jnp.where(lane == 4 * j + chn,
                                     chn * 65536 + base, gidx)
            idx_v[...] = gidx
            pltpu.async_copy(cls_hbm.at[idx_v], gat_v, sem).wait()
            gv = gat_v[...]

            ids = []
            for j in range(3):
                best = gv[4 * j]
                cid = jnp.int32(0)
                for chn in range(1, 4):
                    vc = gv[4 * j + chn]
                    take = vc > best
                    cid = jnp.where(take, jnp.int32(chn), cid)
                    best = jnp.where(take, vc, best)
                ids.append(cid)
            total = ids[0] + ids[1] + ids[2]
            ids = [jnp.where(ids[j] == 3, 6 - total, ids[j]) for j in range(3)]

            ranks = []
            for kk in range(3):
                rk = jnp.int32(0)
                for j in range(3):
                    if j == kk:
                        continue
                    lt = (ids[j] < ids[kk]) | ((ids[j] == ids[kk]) & (j < kk))
                    rk = rk + lt.astype(jnp.int32)
                ranks.append(rk)

            def pick(p, vv):
                return jnp.where(
                    ranks[0] == p, vv[0],
                    jnp.where(ranks[1] == p, vv[1], vv[2]))

            cf = [c[j].astype(jnp.float32) for j in range(3)]
            rf = [r[j].astype(jnp.float32) for j in range(3)]
            xs = [pick(p, cf) for p in range(3)]
            ys = [pick(p, rf) for p in range(3)]
            so = [pick(p, scores) for p in range(3)]

            A = (xs[1] * ys[2] - xs[2] * ys[1]
                 - xs[0] * ys[2] + xs[2] * ys[0]
                 + xs[0] * ys[1] - xs[1] * ys[0])
            swap = A > 0
            fx = [jnp.where(swap, xs[1], xs[0]),
                  jnp.where(swap, xs[0], xs[1]), xs[2]]
            fy = [jnp.where(swap, ys[1], ys[0]),
                  jnp.where(swap, ys[0], ys[1]), ys[2]]
            ov = jnp.zeros((16,), jnp.float32)
            for p in range(3):
                ov = jnp.where(lane == 2 * p, fx[p], ov)
                ov = jnp.where(lane == 2 * p + 1, fy[p], ov)
                ov = jnp.where(lane == 6 + p, so[p], ov)
            out_v[...] = ov
            pltpu.sync_copy(out_v, out_hbm)

    return k(top16, cls_flat)


def kernel(out_det, out_cls):
    det = out_det[0]
    det_p = jnp.pad(det, ((0, 0), (0, STRIP), (0, 0)))
    res = _detector(det_p)
    fin = _sc_finish(res[0, :16], out_cls[0].reshape(-1))
    kp_xy = fin[:6].reshape(3, 2)
    top_scores = fin[6:9]
    return kp_xy, top_scores


# speedup vs baseline: 1.2202x; 1.2202x over previous
import jax
import jax.numpy as jnp
from jax.experimental import pallas as pl
from jax.experimental.pallas import tpu as pltpu

CELL = 8
THRESH = 0.015
NEG = float("-inf")
BIG = 3e7
SENT = -3e38

STRIP = 64
HALO = 3
TILE = STRIP + 2 * HALO
NSTRIPS = 256 // STRIP


def _pool_y(x, op, padval):
    T = x.shape[2]
    P = [x[0]]
    for p in range(1, 8):
        P.append(op(P[-1], x[p]))
    S = [None] * 8
    S[7] = x[7]
    for p in range(6, -1, -1):
        S[p] = op(S[p + 1], x[p])
    pad = jnp.full((8, 1, 256), padval, x.dtype)

    def up(a):
        return jnp.concatenate([pad, a[:, : T - 1, :]], axis=1)

    def down(a):
        return jnp.concatenate([a[:, 1:, :], pad], axis=1)

    outs = [op(P[cy + 4], up(S[cy + 4])) for cy in range(4)]
    outs += [op(S[cy - 4], down(P[cy - 4])) for cy in range(4, 8)]
    return jnp.stack(outs, axis=0)


def _pool_x(x, op, padval):
    T = x.shape[2]
    P = [x[:, 0]]
    for p in range(1, 8):
        P.append(op(P[-1], x[:, p]))
    S = [None] * 8
    S[7] = x[:, 7]
    for p in range(6, -1, -1):
        S[p] = op(S[p + 1], x[:, p])
    pad = jnp.full((8, T, 1), padval, x.dtype)

    def left(a):
        return jnp.concatenate([pad, a[:, :, :-1]], axis=2)

    def right(a):
        return jnp.concatenate([a[:, :, 1:], pad], axis=2)

    outs = [op(P[cx + 4], left(S[cx + 4])) for cx in range(4)]
    outs += [op(S[cx - 4], right(P[cx - 4])) for cx in range(4, 8)]
    return jnp.stack(outs, axis=1)


def _pool9(x):
    return _pool_x(_pool_y(x, jnp.maximum, NEG), jnp.maximum, NEG)


def _dilate9(m):
    mb = m.astype(jnp.bfloat16)
    return _pool_x(_pool_y(mb, jnp.maximum, NEG), jnp.maximum, NEG) > 0


def _body(b_ref, c_ref, cls_ref, out_ref, tail_ref, sm_s, sm_i):
    i = pl.program_id(0)

    @pl.when(i == 0)
    def _():
        tail_ref[...] = jnp.zeros_like(tail_ref)

    xa = tail_ref[...]
    xb = b_ref[...]
    xc = c_ref[:, :HALO, :]
    x = jnp.concatenate([xa, xb, xc], axis=1)
    tail_ref[...] = b_ref[:, STRIP - HALO :, :]
    mx = jnp.max(x, axis=0, keepdims=True)
    ex = jnp.exp(x - mx)
    denom = jnp.sum(ex, axis=0, keepdims=True)
    probs = ex[:64] / denom
    s = probs.reshape(8, 8, TILE, 256)

    g0 = i * STRIP - HALO
    rowid = jax.lax.broadcasted_iota(jnp.int32, (1, 1, TILE, 256), 2) + g0
    s = jnp.where((rowid >= 0) & (rowid < 256), s, NEG)

    mask = s == _pool9(s)
    for _ in range(2):
        supp = _dilate9(mask)
        supp_scores = jnp.where(supp, 0.0, s)
        new_max = supp_scores == _pool9(supp_scores)
        mask = mask | (new_max & jnp.logical_not(supp))
    nms = jnp.where(mask, s, 0.0)

    core = nms[:, :, HALO : HALO + STRIP, :]
    vals = jnp.where(core > THRESH, core, SENT)
    groups = []
    for hy in range(2):
        for hx in range(2):
            bv = bi = None
            for cy in range(hy * 4, hy * 4 + 4):
                for cx in range(hx * 4, hx * 4 + 4):
                    v = vals[cy, cx]
                    iconst = jnp.float32(cy * 2048 + cx)
                    if bv is None:
                        bv, bi = v, jnp.full_like(v, iconst)
                    else:
                        gt = v > bv
                        bv = jnp.where(gt, v, bv)
                        bi = jnp.where(gt, iconst, bi)
            groups.append((bv, bi))
    V4 = jnp.stack([g[0] for g in groups], axis=0)
    I4 = jnp.stack([g[1] for g in groups], axis=0)
    ir = jax.lax.broadcasted_iota(jnp.int32, (1, STRIP, 1), 1)
    ik = jax.lax.broadcasted_iota(jnp.int32, (1, 1, 256), 2)
    rowbase = (8 * (STRIP * i + ir) * 2048).astype(jnp.float32)
    colbase = (8 * ik).astype(jnp.float32)
    flatv = I4 + rowbase + colbase
    cand = []
    v = V4
    for _ in range(3):
        m = jnp.max(v)
        eqm = v == m
        sel = jnp.min(jnp.where(eqm, flatv, BIG))
        v = jnp.where(eqm & (flatv == sel), NEG, v)
        cand.append((m, sel))

    @pl.when(i == 0)
    def _():
        for t in range(3):
            sm_s[t] = jnp.float32(NEG)
            sm_i[t] = jnp.float32(BIG)

    pairs = [(sm_s[0], sm_i[0]), (sm_s[1], sm_i[1]), (sm_s[2], sm_i[2])] + cand
    top = []
    cur = pairs
    for _ in range(3):
        bs, bi = cur[0]
        for ss, si in cur[1:]:
            better = (ss > bs) | ((ss == bs) & (si < bi))
            bs = jnp.where(better, ss, bs)
            bi = jnp.where(better, si, bi)
        top.append((bs, bi))
        cur = [
            (
                jnp.where((ss == bs) & (si == bi), jnp.float32(NEG), ss),
                jnp.where((ss == bs) & (si == bi), jnp.float32(BIG), si),
            )
            for ss, si in cur
        ]
    for t in range(3):
        sm_s[t] = top[t][0]
        sm_i[t] = top[t][1]

    @pl.when(i == NSTRIPS - 1)
    def _():
        scores = [top[k][0] for k in range(3)]
        idx = [top[k][1].astype(jnp.int32) for k in range(3)]
        r = [idx[k] // 2048 for k in range(3)]
        c = [idx[k] % 2048 for k in range(3)]

        ir2 = jax.lax.broadcasted_iota(jnp.int32, (256, 256), 0)
        ic2 = jax.lax.broadcasted_iota(jnp.int32, (256, 256), 1)
        ids = []
        for k in range(3):
            oh = (ir2 == r[k] // CELL) & (ic2 == c[k] // CELL)
            best = jnp.max(jnp.where(oh, cls_ref[0], NEG))
            cid = jnp.int32(0)
            for chn in range(1, 4):
                vc = jnp.max(jnp.where(oh, cls_ref[chn], NEG))
                take = vc > best
                cid = jnp.where(take, jnp.int32(chn), cid)
                best = jnp.where(take, vc, best)
            ids.append(cid)

        total = ids[0] + ids[1] + ids[2]
        ids = [jnp.where(ids[k] == 3, 6 - total, ids[k]) for k in range(3)]

        ranks = []
        for k in range(3):
            rk = jnp.int32(0)
            for j in range(3):
                if j == k:
                    continue
                lt = (ids[j] < ids[k]) | ((ids[j] == ids[k]) & (j < k))
                rk = rk + lt.astype(jnp.int32)
            ranks.append(rk)

        def pick(p, vv):
            return jnp.where(
                ranks[0] == p, vv[0], jnp.where(ranks[1] == p, vv[1], vv[2])
            )

        cf = [c[k].astype(jnp.float32) for k in range(3)]
        rf = [r[k].astype(jnp.float32) for k in range(3)]
        xs = [pick(p, cf) for p in range(3)]
        ys = [pick(p, rf) for p in range(3)]
        so = [pick(p, scores) for p in range(3)]
        so = [jnp.where(sp == SENT, jnp.float32(NEG), sp) for sp in so]

        A = (xs[1] * ys[2] - xs[2] * ys[1]
             - xs[0] * ys[2] + xs[2] * ys[0]
             + xs[0] * ys[1] - xs[1] * ys[0])
        swap = A > 0
        fx = [jnp.where(swap, xs[1], xs[0]), jnp.where(swap, xs[0], xs[1]), xs[2]]
        fy = [jnp.where(swap, ys[1], ys[0]), jnp.where(swap, ys[0], ys[1]), ys[2]]

        rI = jax.lax.broadcasted_iota(jnp.int32, (8, 128), 0)
        cI = jax.lax.broadcasted_iota(jnp.int32, (8, 128), 1)
        acc = jnp.zeros((8, 128), jnp.float32)
        entries = [(0, 0, fx[0]), (0, 1, fy[0]),
                   (1, 0, fx[1]), (1, 1, fy[1]),
                   (2, 0, fx[2]), (2, 1, fy[2]),
                   (3, 0, so[0]), (3, 1, so[1]), (3, 2, so[2])]
        for rr, cc, val in entries:
            acc = acc + jnp.where((rI == rr) & (cI == cc), val, 0.0)
        out_ref[...] = acc


def _detector(det_p, cls_):
    return pl.pallas_call(
        _body,
        grid=(NSTRIPS,),
        in_specs=[
            pl.BlockSpec((65, STRIP, 256), lambda i: (0, i, 0)),
            pl.BlockSpec((65, STRIP, 256), lambda i: (0, i + 1, 0)),
            pl.BlockSpec((4, 256, 256), lambda i: (0, 0, 0)),
        ],
        out_specs=pl.BlockSpec((8, 128), lambda i: (0, 0)),
        out_shape=jax.ShapeDtypeStruct((8, 128), jnp.float32),
        scratch_shapes=[
            pltpu.VMEM((65, HALO, 256), jnp.float32),
            pltpu.SMEM((8,), jnp.float32),
            pltpu.SMEM((8,), jnp.float32),
        ],
    )(det_p, det_p, cls_)


def kernel(out_det, out_cls):
    det = out_det[0]
    det_p = jnp.pad(det, ((0, 0), (0, STRIP), (0, 0)))
    res = _detector(det_p, out_cls[0])
    kp_xy = res[:3, :2]
    top_scores = res[3, :3]
    return kp_xy, top_scores
